# Initial kernel scaffold; baseline (speedup 1.0000x reference)
import jax
import jax.numpy as jnp
from jax.experimental import pallas as pl


def kernel(boxes, scores):
    raise NotImplementedError("write your pallas kernel here")


# speedup vs baseline: 13.5019x; 13.5019x over previous
import jax
import jax.numpy as jnp
from jax.experimental import pallas as pl

_N = 20000
_PADN = 20480
_R = 160
_C = 128
_K = 1000
_DET = 100
_OUTR = 104


def _nms_body(score_ref, box_ref, out_ref):
    raw = score_ref[...]
    s = jnp.where(raw > 0.05, raw, 0.0)
    x1 = box_ref[0]
    y1 = box_ref[1]
    x2 = box_ref[2]
    y2 = box_ref[3]
    areas = (x2 - x1) * (y2 - y1)

    row_i = jax.lax.broadcasted_iota(jnp.int32, (_R, _C), 0)
    col_i = jax.lax.broadcasted_iota(jnp.int32, (_R, _C), 1)
    lin = row_i * _C + col_i

    sbits = jax.lax.bitcast_convert_type(s, jnp.int32)

    def bs_body(_, lh):
        lo, hi = lh
        mid = lo + ((hi - lo + 1) // 2)
        cnt = jnp.sum((sbits >= mid).astype(jnp.int32))
        ge = cnt >= _K
        return jnp.where(ge, mid, lo), jnp.where(ge, hi, mid - 1)

    t_bits, _ = jax.lax.fori_loop(
        0, 31, bs_body, (jnp.int32(0), jnp.int32(0x7F7FFFFF))
    )

    cnt_gt = jnp.sum((sbits > t_bits).astype(jnp.int32))
    m = (_K - cnt_gt).astype(jnp.float32)
    eq = sbits == t_bits
    eqf = eq.astype(jnp.float32)

    mrow = (
        jax.lax.broadcasted_iota(jnp.int32, (_C, _C), 0)
        < jax.lax.broadcasted_iota(jnp.int32, (_C, _C), 1)
    ).astype(jnp.float32)
    prow = jnp.dot(eqf, mrow, preferred_element_type=jnp.float32)
    rs = jnp.sum(eqf, axis=1, keepdims=True)
    mrows = (
        jax.lax.broadcasted_iota(jnp.int32, (_R, _R), 1)
        < jax.lax.broadcasted_iota(jnp.int32, (_R, _R), 0)
    ).astype(jnp.float32)
    roff = jnp.dot(mrows, rs, preferred_element_type=jnp.float32)
    excl = prow + roff

    mask = (sbits > t_bits) | (eq & (excl < m))
    w0 = jnp.where(mask, s, 0.0)

    lane = jax.lax.broadcasted_iota(jnp.int32, (1, _C), 1)

    def body(i, w):
        best = jnp.max(w)
        idx = jnp.min(jnp.where(w == best, lin, jnp.int32(1 << 30)))
        is_best = lin == idx
        bx1 = jnp.sum(jnp.where(is_best, x1, 0.0))
        by1 = jnp.sum(jnp.where(is_best, y1, 0.0))
        bx2 = jnp.sum(jnp.where(is_best, x2, 0.0))
        by2 = jnp.sum(jnp.where(is_best, y2, 0.0))
        barea = (bx2 - bx1) * (by2 - by1)
        iw = jnp.maximum(jnp.minimum(bx2, x2) - jnp.maximum(bx1, x1), 0.0)
        ih = jnp.maximum(jnp.minimum(by2, y2) - jnp.maximum(by1, y1), 0.0)
        inter = iw * ih
        union = jnp.maximum(barea + areas - inter, 1e-8)
        iou = inter / union
        w = jnp.where((iou > 0.5) | is_best, 0.0, w)
        valid = jnp.where(best > 0.0, 1.0, 0.0)
        row = jnp.where(lane == 0, bx1 * valid, 0.0)
        row = jnp.where(lane == 1, by1 * valid, row)
        row = jnp.where(lane == 2, bx2 * valid, row)
        row = jnp.where(lane == 3, by2 * valid, row)
        row = jnp.where(lane == 4, best * valid, row)
        out_ref[pl.ds(i, 1), :] = row
        return w

    jax.lax.fori_loop(0, _DET, body, w0)


def _build(interpret=False):
    return pl.pallas_call(
        _nms_body,
        out_shape=jax.ShapeDtypeStruct((_OUTR, _C), jnp.float32),
        interpret=interpret,
    )


@jax.jit
def kernel(boxes, scores):
    s = jnp.pad(scores, (0, _PADN - _N)).reshape(_R, _C)
    b = jnp.pad(boxes, ((0, _PADN - _N), (0, 0))).T.reshape(4, _R, _C)
    out = _build()(s, b)
    return out[:_DET, :5]
